# baseline (speedup 1.0000x reference)
import functools

import jax
import jax.numpy as jnp
from jax import lax
from jax.experimental import pallas as pl
from jax.experimental.pallas import tpu as pltpu
from jax.experimental.pallas import tpu_sc as plsc

N = 10000
E = 320000
F = 128
K = 5
LAMBDA_MAX = 2.0

NC = 2
NS = 16
NW = NC * NS
C = 128
NCHUNK = -(-((E + NW * C - 1) // (NW * C)) // 8) * 8
E_PAD = NW * C * NCHUNK
EPW = E_PAD // NW
NCHT = E_PAD // C
N_PAD = 10240
RPT = N_PAD // NS


def _worker_id():
    return lax.axis_index("s") * NC + lax.axis_index("c")


def _deg_body(rc_hbm, ew_hbm, out_hbm, rcall_v, ewall_v, zbd_v, sem,
              acc_sh):
    c = lax.axis_index("c")
    s = lax.axis_index("s")
    w = _worker_id()

    pltpu.sync_copy(rc_hbm.at[pl.ds(w * NCHUNK, NCHUNK)], rcall_v)
    pltpu.sync_copy(ew_hbm.at[pl.ds(w * NCHUNK, NCHUNK)], ewall_v)

    def zloop(i, _):
        zbd_v[pl.ds(i * 16, 16)] = jnp.zeros((16,), jnp.float32)
        return 0
    lax.fori_loop(0, RPT // 16, zloop, 0)
    pltpu.sync_copy(zbd_v, acc_sh.at[pl.ds(s * RPT, RPT)])
    plsc.subcore_barrier()

    QD = 8

    def chunk(i, _):
        pltpu.async_copy(ewall_v.at[i], acc_sh.at[rcall_v.at[i, 0]], sem,
                         add=True)

        @pl.when(i >= QD)
        def _():
            pltpu.make_async_copy(ewall_v.at[0], acc_sh.at[rcall_v.at[0, 0]],
                                  sem).wait()
        return 0
    lax.fori_loop(0, NCHUNK, chunk, 0)
    for _ in range(QD):
        pltpu.make_async_copy(ewall_v.at[0], acc_sh.at[rcall_v.at[0, 0]],
                              sem).wait()
    plsc.subcore_barrier()
    pltpu.sync_copy(acc_sh.at[pl.ds(s * RPT, RPT)],
                    out_hbm.at[c, pl.ds(s * RPT, RPT)])


_deg_call = functools.partial(
    pl.kernel,
    out_type=jax.ShapeDtypeStruct((NC, N_PAD), jnp.float32),
    mesh=plsc.VectorSubcoreMesh(core_axis_name="c", subcore_axis_name="s"),
    compiler_params=pltpu.CompilerParams(needs_layout_passes=False),
    scratch_types=[
        pltpu.VMEM((NCHUNK, 2, C), jnp.int32),
        pltpu.VMEM((NCHUNK, C), jnp.float32),
        pltpu.VMEM((RPT,), jnp.float32),
        pltpu.SemaphoreType.DMA,
        pltpu.VMEM_SHARED((N_PAD,), jnp.float32),
    ],
)(_deg_body)


def _lap_body(rc_hbm, ew_hbm, dis_hbm, lap_hbm, rcall_v, ewall_v, dis_v):
    w = _worker_id()
    pltpu.sync_copy(rc_hbm.at[pl.ds(w * NCHUNK, NCHUNK)], rcall_v)
    pltpu.sync_copy(ew_hbm.at[pl.ds(w * NCHUNK, NCHUNK)], ewall_v)
    pltpu.sync_copy(dis_hbm, dis_v)

    def chunk(i, _):
        for j in range(C // 16):
            sl = pl.ds(j * 16, 16)
            dr = plsc.load_gather(dis_v, [rcall_v[i, 0, sl]])
            dc = plsc.load_gather(dis_v, [rcall_v[i, 1, sl]])
            ewall_v[i, sl] = (-1.0) * dr * ewall_v[i, sl] * dc
        return 0
    lax.fori_loop(0, NCHUNK, chunk, 0)
    pltpu.sync_copy(ewall_v, lap_hbm.at[pl.ds(w * NCHUNK, NCHUNK)])


_lap_call = functools.partial(
    pl.kernel,
    out_type=jax.ShapeDtypeStruct((NCHT, C), jnp.float32),
    mesh=plsc.VectorSubcoreMesh(core_axis_name="c", subcore_axis_name="s"),
    compiler_params=pltpu.CompilerParams(needs_layout_passes=False),
    scratch_types=[
        pltpu.VMEM((NCHUNK, 2, C), jnp.int32),
        pltpu.VMEM((NCHUNK, C), jnp.float32),
        pltpu.VMEM((N_PAD,), jnp.float32),
    ],
)(_lap_body)


B = 1024
EH = E_PAD // NC
NB = EH // B
CAPQ = 2048
SELSZ = CAPQ + 256


def _prop_body(h_hbm, ed_hbm, out_hbm,
               ed_v, selr_v, selc_v, sell_v, rows_v, acc_v,
               sem_b, sem_g):
    c = lax.axis_index("c")
    s = lax.axis_index("s")
    lo = s * RPT

    def z_acc(i, _):
        for j in range(F // 16):
            acc_v[i, pl.ds(j * 16, 16)] = jnp.zeros((16,), jnp.float32)
        return 0
    lax.fori_loop(0, RPT, z_acc, 0)

    def z_sel(i, _):
        sl = pl.ds(i * 16, 16)
        selr_v[sl] = jnp.zeros((16,), jnp.int32)
        selc_v[sl] = jnp.full((16,), 0, jnp.int32) + lo
        sell_v[sl] = jnp.zeros((16,), jnp.int32)
        return 0
    lax.fori_loop(0, SELSZ // 16, z_sel, 0)

    def accum_chunk(q):
        qb = lax.rem(q, 2)

        def grp(g, _):
            base = q * 128 + g * 16
            cl16 = selc_v[pl.ds(base, 16)]
            lw16 = plsc.bitcast(sell_v[pl.ds(base, 16)], jnp.float32)
            for l in range(16):
                sv = lw16[l]
                cl = cl16[l] - lo
                e = g * 16 + l
                for j in range(F // 16):
                    sl = pl.ds(j * 16, 16)
                    plsc.addupdate(acc_v.at[cl, sl], rows_v[qb, e, sl] * sv)
            return 0
        lax.fori_loop(0, 8, grp, 0)

    def start_gather(q):
        pltpu.async_copy(h_hbm.at[selr_v.at[pl.ds(q * 128, 128)]],
                         rows_v.at[lax.rem(q, 2)],
                         sem_g.at[lax.rem(q, 2)])

    def wait_gather(q):
        pltpu.make_async_copy(h_hbm.at[selr_v.at[pl.ds(0, 128)]],
                              rows_v.at[0], sem_g.at[lax.rem(q, 2)]).wait()

    def flush(mc):
        nc = (mc + 127) // 128

        @pl.when(nc > 0)
        def _():
            start_gather(0)

            def chq(q, _):
                @pl.when(q + 1 < nc)
                def _():
                    start_gather(q + 1)
                wait_gather(q)
                accum_chunk(q)
                return 0
            lax.fori_loop(0, nc, chq, 0)

        def z_used(i, _):
            sl = pl.ds(i * 16, 16)
            selc_v[sl] = jnp.full((16,), 0, jnp.int32) + lo
            sell_v[sl] = jnp.zeros((16,), jnp.int32)
            return 0
        lax.fori_loop(0, (mc + 15) // 16, z_used, 0)

    pltpu.async_copy(ed_hbm.at[c * NB], ed_v.at[0], sem_b)

    def block(b, mc):
        bb = lax.rem(b, 2)
        pltpu.make_async_copy(ed_hbm.at[0], ed_v.at[0], sem_b).wait()

        @pl.when(b + 1 < NB)
        def _():
            pltpu.async_copy(ed_hbm.at[c * NB + b + 1],
                             ed_v.at[lax.rem(b + 1, 2)], sem_b)

        def grp(g, mcg):
            sl = pl.ds(g * 16, 16)
            cv = ed_v[bb, 1, sl]
            m = (cv >= lo) & (cv < lo + RPT)
            plsc.store_compressed(selr_v.at[pl.ds(mcg, 16)],
                                  ed_v[bb, 0, sl], mask=m)
            plsc.store_compressed(selc_v.at[pl.ds(mcg, 16)], cv, mask=m)
            plsc.store_compressed(sell_v.at[pl.ds(mcg, 16)],
                                  ed_v[bb, 2, sl], mask=m)
            cnt = plsc.all_reduce_population_count(m)
            return mcg + cnt[0]
        mc = lax.fori_loop(0, B // 16, grp, mc)

        do_flush = mc > CAPQ - B
        @pl.when(do_flush)
        def _():
            flush(mc)
        return jnp.where(do_flush, 0, mc)

    mc = lax.fori_loop(0, NB, block, jnp.int32(0))
    flush(mc)

    pltpu.sync_copy(acc_v, out_hbm.at[c, pl.ds(lo, RPT)])


_prop_call = functools.partial(
    pl.kernel,
    out_type=jax.ShapeDtypeStruct((NC, N_PAD, F), jnp.float32),
    mesh=plsc.VectorSubcoreMesh(core_axis_name="c", subcore_axis_name="s"),
    compiler_params=pltpu.CompilerParams(needs_layout_passes=False),
    scratch_types=[
        pltpu.VMEM((2, 3, B), jnp.int32),
        pltpu.VMEM((SELSZ,), jnp.int32),
        pltpu.VMEM((SELSZ,), jnp.int32),
        pltpu.VMEM((SELSZ,), jnp.int32),
        pltpu.VMEM((2, 128, F), jnp.float32),
        pltpu.VMEM((RPT, F), jnp.float32),
        pltpu.SemaphoreType.DMA,
        pltpu.SemaphoreType.DMA((2,)),
    ],
)(_prop_body)


def _dis_body(deg_ref, out_ref):
    d = deg_ref[0] + deg_ref[1]
    out_ref[...] = jnp.where(d > 0, 1.0 / jnp.sqrt(d), 0.0)


def _dis_call(deg2):
    return pl.pallas_call(
        _dis_body,
        out_shape=jax.ShapeDtypeStruct((N_PAD // 128, 128), jnp.float32),
    )(deg2)


def _combine_body(a, b, p_ref, prev_ref, out_ref):
    out_ref[...] = a * (p_ref[0] + p_ref[1]) - b * prev_ref[...]


def _combine_call(p, prev, a, b):
    blk = 1024
    grid = N_PAD // blk
    return pl.pallas_call(
        functools.partial(_combine_body, a, b),
        grid=(grid,),
        in_specs=[
            pl.BlockSpec((NC, blk, F), lambda i: (0, i, 0)),
            pl.BlockSpec((blk, F), lambda i: (i, 0)),
        ],
        out_specs=pl.BlockSpec((blk, F), lambda i: (i, 0)),
        out_shape=jax.ShapeDtypeStruct((N_PAD, F), jnp.float32),
    )(p, prev)


def _matmul_body(x_ref, w_ref, b_ref, out_ref):
    out_ref[...] = jnp.dot(
        x_ref[...], w_ref[...], preferred_element_type=jnp.float32,
        precision=lax.Precision.HIGHEST) + b_ref[...]


def _matmul_call(xcat, wr, bias):
    blk = 1024
    grid = N_PAD // blk
    return pl.pallas_call(
        _matmul_body,
        grid=(grid,),
        in_specs=[
            pl.BlockSpec((blk, K * F), lambda i: (i, 0)),
            pl.BlockSpec((K * F, F), lambda i: (0, 0)),
            pl.BlockSpec((1, F), lambda i: (0, 0)),
        ],
        out_specs=pl.BlockSpec((blk, F), lambda i: (i, 0)),
        out_shape=jax.ShapeDtypeStruct((N_PAD, F), jnp.float32),
    )(xcat, wr, bias)


def kernel(x, edge_weight, W, bias, edge_index):
    row = jnp.zeros((E_PAD,), jnp.int32).at[:E].set(edge_index[0])
    col = jnp.zeros((E_PAD,), jnp.int32).at[:E].set(edge_index[1])
    ew = jnp.zeros((E_PAD,), jnp.float32).at[:E].set(edge_weight)
    h0 = jnp.zeros((N_PAD, F), jnp.float32).at[:N].set(x)
    rc = jnp.stack([row.reshape(NCHT, C), col.reshape(NCHT, C)], axis=1)
    ew2 = ew.reshape(NCHT, C)

    deg2 = _deg_call(rc, ew2)
    dis = _dis_call(deg2.reshape(NC, N_PAD // 128, 128)).reshape(N_PAD)
    lap = _lap_call(rc, ew2, dis)

    lapbits = lax.bitcast_convert_type(lap.reshape(E_PAD), jnp.int32)
    edata = jnp.stack([row.reshape(NC, NB, B), col.reshape(NC, NB, B),
                       lapbits.reshape(NC, NB, B)],
                      axis=2).reshape(NC * NB, 3, B)

    tx = [h0]
    for k in range(1, K):
        p = _prop_call(tx[-1], edata)
        a, b = (1.0, 0.0) if k == 1 else (2.0, 1.0)
        prev = tx[-1] if k == 1 else tx[-2]
        tx.append(_combine_call(p, prev, a, b))

    xcat = jnp.concatenate(tx, axis=1)
    wr = W.reshape(K * F, F)
    out = _matmul_call(xcat, wr, bias.reshape(1, F))
    return out[:N]


# speedup vs baseline: 2.8212x; 2.8212x over previous
import functools

import jax
import jax.numpy as jnp
from jax import lax
from jax.experimental import pallas as pl
from jax.experimental.pallas import tpu as pltpu
from jax.experimental.pallas import tpu_sc as plsc

N = 10000
E = 320000
F = 128
K = 5
LAMBDA_MAX = 2.0

NC = 2
NS = 16
NW = NC * NS
C = 128
E_PAD = ((E + NW * C - 1) // (NW * C)) * (NW * C)
EPW = E_PAD // NW
NCHUNK = EPW // C
N_PAD = 10240
RPT = N_PAD // NS


def _worker_id():
    return lax.axis_index("s") * NC + lax.axis_index("c")


def _deg_body(rc_hbm, ew_hbm, out_hbm, idx_v, val_v, zb_v, acc_sh):
    c = lax.axis_index("c")
    s = lax.axis_index("s")
    w = _worker_id()

    def zloop(i, _):
        zb_v[pl.ds(i * 16, 16)] = jnp.zeros((16,), jnp.float32)
        return 0
    lax.fori_loop(0, RPT // 16, zloop, 0)
    pltpu.sync_copy(zb_v, acc_sh.at[pl.ds(s * RPT, RPT)])
    plsc.subcore_barrier()

    def chunk(i, _):
        base = w * EPW + i * C
        pltpu.sync_copy(rc_hbm.at[w * NCHUNK + i, 0], idx_v)
        pltpu.sync_copy(ew_hbm.at[pl.ds(base, C)], val_v)
        pltpu.sync_copy(val_v, acc_sh.at[idx_v], add=True)
        return 0
    lax.fori_loop(0, NCHUNK, chunk, 0)
    plsc.subcore_barrier()
    pltpu.sync_copy(acc_sh.at[pl.ds(s * RPT, RPT)], out_hbm.at[c, pl.ds(s * RPT, RPT)])


_deg_call = functools.partial(
    pl.kernel,
    out_type=jax.ShapeDtypeStruct((NC, N_PAD), jnp.float32),
    mesh=plsc.VectorSubcoreMesh(core_axis_name="c", subcore_axis_name="s"),
    compiler_params=pltpu.CompilerParams(needs_layout_passes=False),
    scratch_types=[
        pltpu.VMEM((C,), jnp.int32),
        pltpu.VMEM((C,), jnp.float32),
        pltpu.VMEM((RPT,), jnp.float32),
        pltpu.VMEM_SHARED((N_PAD,), jnp.float32),
    ],
)(_deg_body)


def _lap_body(rc_hbm, ew_hbm, dis_hbm, lap_hbm,
              ridx_v, cidx_v, ew_v, lw_v, dis_v):
    w = _worker_id()
    pltpu.sync_copy(dis_hbm, dis_v)

    def chunk(i, _):
        base = w * EPW + i * C
        pltpu.sync_copy(rc_hbm.at[w * NCHUNK + i, 0], ridx_v)
        pltpu.sync_copy(rc_hbm.at[w * NCHUNK + i, 1], cidx_v)
        pltpu.sync_copy(ew_hbm.at[pl.ds(base, C)], ew_v)
        for j in range(C // 16):
            sl = pl.ds(j * 16, 16)
            dr = plsc.load_gather(dis_v, [ridx_v[sl]])
            dc = plsc.load_gather(dis_v, [cidx_v[sl]])
            lw_v[sl] = (-1.0) * dr * ew_v[sl] * dc
        pltpu.sync_copy(lw_v, lap_hbm.at[pl.ds(base, C)])
        return 0
    lax.fori_loop(0, NCHUNK, chunk, 0)


_lap_call = functools.partial(
    pl.kernel,
    out_type=jax.ShapeDtypeStruct((E_PAD,), jnp.float32),
    mesh=plsc.VectorSubcoreMesh(core_axis_name="c", subcore_axis_name="s"),
    compiler_params=pltpu.CompilerParams(needs_layout_passes=False),
    scratch_types=[
        pltpu.VMEM((C,), jnp.int32),
        pltpu.VMEM((C,), jnp.int32),
        pltpu.VMEM((C,), jnp.float32),
        pltpu.VMEM((C,), jnp.float32),
        pltpu.VMEM((N_PAD,), jnp.float32),
    ],
)(_lap_body)


def _prop_body(hb_hbm, rc_hbm, lap_hbm, out_hbm,
               rc_v, lw_v, rows_bf, rows_f, sem_g, sem_s, sem_i, acc_sh):
    c = lax.axis_index("c")
    s = lax.axis_index("s")
    w = _worker_id()

    def zloop(i, _):
        for j in range(F // 16):
            rows_f[i, pl.ds(j * 16, 16)] = jnp.zeros((16,), jnp.float32)
        return 0
    lax.fori_loop(0, C, zloop, 0)
    for q in range(RPT // C):
        pltpu.sync_copy(rows_f, acc_sh.at[pl.ds(s * RPT + q * C, C)])
    plsc.subcore_barrier()

    cbase = w * NCHUNK

    def load_idx(i):
        pltpu.async_copy(rc_hbm.at[cbase + i], rc_v.at[i % 4], sem_i)
        pltpu.async_copy(lap_hbm.at[cbase + i], lw_v.at[i % 4], sem_i)

    def wait_idx():
        pltpu.make_async_copy(rc_hbm.at[0], rc_v.at[0], sem_i).wait()
        pltpu.make_async_copy(lap_hbm.at[0], lw_v.at[0], sem_i).wait()

    def start_gather(i):
        pltpu.async_copy(hb_hbm.at[rc_v.at[i % 4, 0]], rows_bf.at[i % 2],
                         sem_g)

    def wait_gather():
        pltpu.make_async_copy(hb_hbm.at[rc_v.at[0, 0]], rows_bf.at[0],
                              sem_g).wait()

    def scale(i):
        b = i % 2
        i4 = i % 4

        def body(g, _):
            lw16 = lw_v[i4, pl.ds(g * 16, 16)]
            for l in range(16):
                e = g * 16 + l
                sv = lw16[l]
                for j in range(F // 32):
                    vi = rows_bf[b, e, pl.ds(j * 16, 16)]
                    v32 = plsc.bitcast(vi, jnp.bfloat16)
                    va, vb = plsc.unpack(
                        v32, format=plsc.PackFormat.INTERLEAVED)
                    rows_f[e, pl.ds(j * 32, 16)] = va * sv
                    rows_f[e, pl.ds(j * 32 + 16, 16)] = vb * sv
            return 0
        lax.fori_loop(0, C // 16, body, 0)

    def start_scatter(i):
        pltpu.async_copy(rows_f, acc_sh.at[rc_v.at[i % 4, 1]],
                         sem_s, add=True)

    def wait_scatter():
        pltpu.make_async_copy(rows_f, acc_sh.at[rc_v.at[0, 1]],
                              sem_s).wait()

    load_idx(0)
    wait_idx()
    start_gather(0)
    load_idx(1)
    wait_idx()
    wait_gather()
    start_gather(1)
    load_idx(2)
    scale(0)
    start_scatter(0)

    def steady(i, _):
        wait_idx()
        wait_gather()
        start_gather(i + 1)
        load_idx(i + 2)
        wait_scatter()
        scale(i)
        start_scatter(i)
        return 0
    lax.fori_loop(1, NCHUNK - 2, steady, 0)

    wait_idx()
    wait_gather()
    start_gather(NCHUNK - 1)
    wait_scatter()
    scale(NCHUNK - 2)
    start_scatter(NCHUNK - 2)
    wait_gather()
    wait_scatter()
    scale(NCHUNK - 1)
    start_scatter(NCHUNK - 1)
    wait_scatter()

    plsc.subcore_barrier()
    pltpu.sync_copy(acc_sh.at[pl.ds(s * RPT, RPT)],
                    out_hbm.at[c, pl.ds(s * RPT, RPT)])


_prop_call = functools.partial(
    pl.kernel,
    out_type=jax.ShapeDtypeStruct((NC, N_PAD, F), jnp.float32),
    mesh=plsc.VectorSubcoreMesh(core_axis_name="c", subcore_axis_name="s"),
    compiler_params=pltpu.CompilerParams(needs_layout_passes=False,
                                         use_tc_tiling_on_sc=False),
    scratch_types=[
        pltpu.VMEM((4, 2, C), jnp.int32),
        pltpu.VMEM((4, C), jnp.float32),
        pltpu.VMEM((2, C, F // 2), jnp.int32),
        pltpu.VMEM((C, F), jnp.float32),
        pltpu.SemaphoreType.DMA,
        pltpu.SemaphoreType.DMA,
        pltpu.SemaphoreType.DMA,
        pltpu.VMEM_SHARED((N_PAD, F), jnp.float32),
    ],
)(_prop_body)


def _dis_body(deg_ref, out_ref):
    d = deg_ref[0] + deg_ref[1]
    out_ref[...] = jnp.where(d > 0, 1.0 / jnp.sqrt(d), 0.0)


def _dis_call(deg2):
    return pl.pallas_call(
        _dis_body,
        out_shape=jax.ShapeDtypeStruct((N_PAD // 128, 128), jnp.float32),
    )(deg2)


def _combine_body(a, b, blk, p_ref, prev_ref, out_ref, outbf_ref):
    po = p_ref[0] + p_ref[1]
    po = po.reshape(blk, F // 32, 2, 16).transpose(0, 1, 3, 2).reshape(blk, F)
    t = a * po - b * prev_ref[...]
    out_ref[...] = t
    outbf_ref[...] = t.astype(jnp.bfloat16)


def _combine_call(p, prev, a, b):
    blk = 1024
    grid = N_PAD // blk
    return pl.pallas_call(
        functools.partial(_combine_body, a, b, blk),
        grid=(grid,),
        in_specs=[
            pl.BlockSpec((NC, blk, F), lambda i: (0, i, 0)),
            pl.BlockSpec((blk, F), lambda i: (i, 0)),
        ],
        out_specs=[pl.BlockSpec((blk, F), lambda i: (i, 0)),
                   pl.BlockSpec((blk, F), lambda i: (i, 0))],
        out_shape=[jax.ShapeDtypeStruct((N_PAD, F), jnp.float32),
                   jax.ShapeDtypeStruct((N_PAD, F), jnp.bfloat16)],
    )(p, prev)


def _matmul_body(x_ref, w_ref, b_ref, out_ref):
    out_ref[...] = jnp.dot(
        x_ref[...], w_ref[...], preferred_element_type=jnp.float32,
        precision=lax.Precision.HIGHEST) + b_ref[...]


def _matmul_call(xcat, wr, bias):
    blk = 1024
    grid = N_PAD // blk
    return pl.pallas_call(
        _matmul_body,
        grid=(grid,),
        in_specs=[
            pl.BlockSpec((blk, K * F), lambda i: (i, 0)),
            pl.BlockSpec((K * F, F), lambda i: (0, 0)),
            pl.BlockSpec((1, F), lambda i: (0, 0)),
        ],
        out_specs=pl.BlockSpec((blk, F), lambda i: (i, 0)),
        out_shape=jax.ShapeDtypeStruct((N_PAD, F), jnp.float32),
    )(xcat, wr, bias)


def kernel(x, edge_weight, W, bias, edge_index):
    row = jnp.zeros((E_PAD,), jnp.int32).at[:E].set(edge_index[0])
    col = jnp.zeros((E_PAD,), jnp.int32).at[:E].set(edge_index[1])
    ew = jnp.zeros((E_PAD,), jnp.float32).at[:E].set(edge_weight)
    h0 = jnp.zeros((N_PAD, F), jnp.float32).at[:N].set(x)
    ncht = E_PAD // C
    rc = jnp.stack([row.reshape(ncht, C), col.reshape(ncht, C)], axis=1)

    deg2 = _deg_call(rc, ew)
    dis = _dis_call(deg2.reshape(NC, N_PAD // 128, 128)).reshape(N_PAD)
    lap = _lap_call(rc, ew, dis).reshape(ncht, C)

    def to_i32(tbf):
        return lax.bitcast_convert_type(
            tbf.reshape(N_PAD, F // 2, 2), jnp.int32)

    tx = [h0]
    tb = [to_i32(h0.astype(jnp.bfloat16))]
    for k in range(1, K):
        p = _prop_call(tb[-1], rc, lap)
        a, b = (1.0, 0.0) if k == 1 else (2.0, 1.0)
        prev = tx[-1] if k == 1 else tx[-2]
        t32, tbf = _combine_call(p, prev, a, b)
        tx.append(t32)
        tb.append(to_i32(tbf))

    xcat = jnp.concatenate(tx, axis=1)
    wr = W.reshape(K * F, F)
    out = _matmul_call(xcat, wr, bias.reshape(1, F))
    return out[:N]


# speedup vs baseline: 2.8227x; 1.0005x over previous
import functools

import jax
import jax.numpy as jnp
from jax import lax
from jax.experimental import pallas as pl
from jax.experimental.pallas import tpu as pltpu
from jax.experimental.pallas import tpu_sc as plsc

N = 10000
E = 320000
F = 128
K = 5
LAMBDA_MAX = 2.0

NC = 2
NS = 16
NW = NC * NS
C = 128
E_PAD = ((E + NW * C - 1) // (NW * C)) * (NW * C)
EPW = E_PAD // NW
NCHUNK = EPW // C
N_PAD = 10240
RPT = N_PAD // NS


def _worker_id():
    return lax.axis_index("s") * NC + lax.axis_index("c")


def _deg_body(rc_hbm, ew_hbm, out_hbm, idx_v, val_v, zb_v, acc_sh):
    c = lax.axis_index("c")
    s = lax.axis_index("s")
    w = _worker_id()

    def zloop(i, _):
        zb_v[pl.ds(i * 16, 16)] = jnp.zeros((16,), jnp.float32)
        return 0
    lax.fori_loop(0, RPT // 16, zloop, 0)
    pltpu.sync_copy(zb_v, acc_sh.at[pl.ds(s * RPT, RPT)])
    plsc.subcore_barrier()

    def chunk(i, _):
        base = w * EPW + i * C
        pltpu.sync_copy(rc_hbm.at[w * NCHUNK + i, 0], idx_v)
        pltpu.sync_copy(ew_hbm.at[pl.ds(base, C)], val_v)
        pltpu.sync_copy(val_v, acc_sh.at[idx_v], add=True)
        return 0
    lax.fori_loop(0, NCHUNK, chunk, 0)
    plsc.subcore_barrier()
    pltpu.sync_copy(acc_sh.at[pl.ds(s * RPT, RPT)], out_hbm.at[c, pl.ds(s * RPT, RPT)])


_deg_call = functools.partial(
    pl.kernel,
    out_type=jax.ShapeDtypeStruct((NC, N_PAD), jnp.float32),
    mesh=plsc.VectorSubcoreMesh(core_axis_name="c", subcore_axis_name="s"),
    compiler_params=pltpu.CompilerParams(needs_layout_passes=False),
    scratch_types=[
        pltpu.VMEM((C,), jnp.int32),
        pltpu.VMEM((C,), jnp.float32),
        pltpu.VMEM((RPT,), jnp.float32),
        pltpu.VMEM_SHARED((N_PAD,), jnp.float32),
    ],
)(_deg_body)


def _lap_body(rc_hbm, ew_hbm, dis_hbm, lap_hbm,
              ridx_v, cidx_v, ew_v, lw_v, dis_v):
    w = _worker_id()
    pltpu.sync_copy(dis_hbm, dis_v)

    def chunk(i, _):
        base = w * EPW + i * C
        pltpu.sync_copy(rc_hbm.at[w * NCHUNK + i, 0], ridx_v)
        pltpu.sync_copy(rc_hbm.at[w * NCHUNK + i, 1], cidx_v)
        pltpu.sync_copy(ew_hbm.at[pl.ds(base, C)], ew_v)
        for j in range(C // 16):
            sl = pl.ds(j * 16, 16)
            dr = plsc.load_gather(dis_v, [ridx_v[sl]])
            dc = plsc.load_gather(dis_v, [cidx_v[sl]])
            lw_v[sl] = (-1.0) * dr * ew_v[sl] * dc
        pltpu.sync_copy(lw_v, lap_hbm.at[pl.ds(base, C)])
        return 0
    lax.fori_loop(0, NCHUNK, chunk, 0)


_lap_call = functools.partial(
    pl.kernel,
    out_type=jax.ShapeDtypeStruct((E_PAD,), jnp.float32),
    mesh=plsc.VectorSubcoreMesh(core_axis_name="c", subcore_axis_name="s"),
    compiler_params=pltpu.CompilerParams(needs_layout_passes=False),
    scratch_types=[
        pltpu.VMEM((C,), jnp.int32),
        pltpu.VMEM((C,), jnp.int32),
        pltpu.VMEM((C,), jnp.float32),
        pltpu.VMEM((C,), jnp.float32),
        pltpu.VMEM((N_PAD,), jnp.float32),
    ],
)(_lap_body)


def _prop_body(hb_hbm, rc_hbm, lap_hbm, out_hbm,
               rc_v, lw_v, rows_bf, rows_f, sem_g, sem_s, sem_i, acc_sh):
    c = lax.axis_index("c")
    s = lax.axis_index("s")
    w = _worker_id()

    def zloop(i, _):
        for j in range(F // 16):
            rows_f[i, pl.ds(j * 16, 16)] = jnp.zeros((16,), jnp.float32)
        return 0
    lax.fori_loop(0, C, zloop, 0)
    for q in range(RPT // C):
        pltpu.sync_copy(rows_f, acc_sh.at[pl.ds(s * RPT + q * C, C)])
    plsc.subcore_barrier()

    cbase = w * NCHUNK

    def load_idx(i):
        pltpu.async_copy(rc_hbm.at[cbase + i], rc_v.at[i % 4], sem_i)
        pltpu.async_copy(lap_hbm.at[cbase + i], lw_v.at[i % 4], sem_i)

    def wait_idx():
        pltpu.make_async_copy(rc_hbm.at[0], rc_v.at[0], sem_i).wait()
        pltpu.make_async_copy(lap_hbm.at[0], lw_v.at[0], sem_i).wait()

    def start_gather(i):
        pltpu.async_copy(hb_hbm.at[rc_v.at[i % 4, 0]],
                         rows_bf.at[lax.rem(i, 3)],
                         sem_g.at[lax.rem(i, 3)])

    def wait_gather(i):
        pltpu.make_async_copy(hb_hbm.at[rc_v.at[0, 0]], rows_bf.at[0],
                              sem_g.at[lax.rem(i, 3)]).wait()

    def scale(i):
        b = lax.rem(i, 3)
        i4 = i % 4

        def body(g, _):
            lw16 = lw_v[i4, pl.ds(g * 16, 16)]
            for l in range(16):
                e = g * 16 + l
                sv = lw16[l]
                for j in range(F // 32):
                    vi = rows_bf[b, e, pl.ds(j * 16, 16)]
                    v32 = plsc.bitcast(vi, jnp.bfloat16)
                    va, vb = plsc.unpack(
                        v32, format=plsc.PackFormat.INTERLEAVED)
                    rows_f[e, pl.ds(j * 32, 16)] = va * sv
                    rows_f[e, pl.ds(j * 32 + 16, 16)] = vb * sv
            return 0
        lax.fori_loop(0, C // 16, body, 0)

    def start_scatter(i):
        pltpu.async_copy(rows_f, acc_sh.at[rc_v.at[i % 4, 1]],
                         sem_s, add=True)

    def wait_scatter():
        pltpu.make_async_copy(rows_f, acc_sh.at[rc_v.at[0, 1]],
                              sem_s).wait()

    load_idx(0)
    wait_idx()
    start_gather(0)
    load_idx(1)
    wait_idx()
    start_gather(1)
    load_idx(2)
    wait_idx()
    start_gather(2)
    wait_gather(0)
    load_idx(3)
    scale(0)
    start_scatter(0)
    wait_idx()
    start_gather(3)
    wait_gather(1)
    wait_scatter()
    load_idx(4)
    scale(1)
    start_scatter(1)

    def steady(i, _):
        wait_idx()
        start_gather(i + 2)
        wait_gather(i)
        wait_scatter()
        load_idx(i + 3)
        scale(i)
        start_scatter(i)
        return 0
    lax.fori_loop(2, NCHUNK - 3, steady, 0)

    wait_idx()
    start_gather(NCHUNK - 1)
    wait_gather(NCHUNK - 3)
    wait_scatter()
    scale(NCHUNK - 3)
    start_scatter(NCHUNK - 3)
    wait_gather(NCHUNK - 2)
    wait_scatter()
    scale(NCHUNK - 2)
    start_scatter(NCHUNK - 2)
    wait_gather(NCHUNK - 1)
    wait_scatter()
    scale(NCHUNK - 1)
    start_scatter(NCHUNK - 1)
    wait_scatter()

    plsc.subcore_barrier()
    pltpu.sync_copy(acc_sh.at[pl.ds(s * RPT, RPT)],
                    out_hbm.at[c, pl.ds(s * RPT, RPT)])


_prop_call = functools.partial(
    pl.kernel,
    out_type=jax.ShapeDtypeStruct((NC, N_PAD, F), jnp.float32),
    mesh=plsc.VectorSubcoreMesh(core_axis_name="c", subcore_axis_name="s"),
    compiler_params=pltpu.CompilerParams(needs_layout_passes=False,
                                         use_tc_tiling_on_sc=False),
    scratch_types=[
        pltpu.VMEM((4, 2, C), jnp.int32),
        pltpu.VMEM((4, C), jnp.float32),
        pltpu.VMEM((3, C, F // 2), jnp.int32),
        pltpu.VMEM((C, F), jnp.float32),
        pltpu.SemaphoreType.DMA((3,)),
        pltpu.SemaphoreType.DMA,
        pltpu.SemaphoreType.DMA,
        pltpu.VMEM_SHARED((N_PAD, F), jnp.float32),
    ],
)(_prop_body)


def _dis_body(deg_ref, out_ref):
    d = deg_ref[0] + deg_ref[1]
    out_ref[...] = jnp.where(d > 0, 1.0 / jnp.sqrt(d), 0.0)


def _dis_call(deg2):
    return pl.pallas_call(
        _dis_body,
        out_shape=jax.ShapeDtypeStruct((N_PAD // 128, 128), jnp.float32),
    )(deg2)


def _combine_body(a, b, blk, p_ref, prev_ref, out_ref, outbf_ref):
    po = p_ref[0] + p_ref[1]
    po = po.reshape(blk, F // 32, 2, 16).transpose(0, 1, 3, 2).reshape(blk, F)
    t = a * po - b * prev_ref[...]
    out_ref[...] = t
    outbf_ref[...] = t.astype(jnp.bfloat16)


def _combine_call(p, prev, a, b):
    blk = 1024
    grid = N_PAD // blk
    return pl.pallas_call(
        functools.partial(_combine_body, a, b, blk),
        grid=(grid,),
        in_specs=[
            pl.BlockSpec((NC, blk, F), lambda i: (0, i, 0)),
            pl.BlockSpec((blk, F), lambda i: (i, 0)),
        ],
        out_specs=[pl.BlockSpec((blk, F), lambda i: (i, 0)),
                   pl.BlockSpec((blk, F), lambda i: (i, 0))],
        out_shape=[jax.ShapeDtypeStruct((N_PAD, F), jnp.float32),
                   jax.ShapeDtypeStruct((N_PAD, F), jnp.bfloat16)],
    )(p, prev)


def _matmul_body(x_ref, w_ref, b_ref, out_ref):
    out_ref[...] = jnp.dot(
        x_ref[...], w_ref[...], preferred_element_type=jnp.float32,
        precision=lax.Precision.HIGHEST) + b_ref[...]


def _matmul_call(xcat, wr, bias):
    blk = 1024
    grid = N_PAD // blk
    return pl.pallas_call(
        _matmul_body,
        grid=(grid,),
        in_specs=[
            pl.BlockSpec((blk, K * F), lambda i: (i, 0)),
            pl.BlockSpec((K * F, F), lambda i: (0, 0)),
            pl.BlockSpec((1, F), lambda i: (0, 0)),
        ],
        out_specs=pl.BlockSpec((blk, F), lambda i: (i, 0)),
        out_shape=jax.ShapeDtypeStruct((N_PAD, F), jnp.float32),
    )(xcat, wr, bias)


def kernel(x, edge_weight, W, bias, edge_index):
    row = jnp.zeros((E_PAD,), jnp.int32).at[:E].set(edge_index[0])
    col = jnp.zeros((E_PAD,), jnp.int32).at[:E].set(edge_index[1])
    ew = jnp.zeros((E_PAD,), jnp.float32).at[:E].set(edge_weight)
    h0 = jnp.zeros((N_PAD, F), jnp.float32).at[:N].set(x)
    ncht = E_PAD // C
    rc = jnp.stack([row.reshape(ncht, C), col.reshape(ncht, C)], axis=1)

    deg2 = _deg_call(rc, ew)
    dis = _dis_call(deg2.reshape(NC, N_PAD // 128, 128)).reshape(N_PAD)
    lap = _lap_call(rc, ew, dis).reshape(ncht, C)

    def to_i32(tbf):
        return lax.bitcast_convert_type(
            tbf.reshape(N_PAD, F // 2, 2), jnp.int32)

    tx = [h0]
    tb = [to_i32(h0.astype(jnp.bfloat16))]
    for k in range(1, K):
        p = _prop_call(tb[-1], rc, lap)
        a, b = (1.0, 0.0) if k == 1 else (2.0, 1.0)
        prev = tx[-1] if k == 1 else tx[-2]
        t32, tbf = _combine_call(p, prev, a, b)
        tx.append(t32)
        tb.append(to_i32(tbf))

    xcat = jnp.concatenate(tx, axis=1)
    wr = W.reshape(K * F, F)
    out = _matmul_call(xcat, wr, bias.reshape(1, F))
    return out[:N]
